# baseline (device time: 64296 ns/iter reference)
import jax
import jax.numpy as jnp
from jax import lax
from jax.experimental import pallas as pl
from jax.experimental.pallas import tpu as pltpu

N_DEV = 4
N_BLK = 4
SCALE = 0.08838834764831843
HQ, HKV, DH = 8, 2, 128
GQA = HQ // HKV


def kernel(x, Wq, Wo, K_ext, V_ext):
    _, sq, d = x.shape
    sqb = sq // N_BLK
    f32, bf16 = jnp.float32, jnp.bfloat16
    ROWS = [0, 2 * sqb, sqb, 3 * sqb]

    def body(x_ref, wq_ref, wo_ref, k_ref, v_ref, out_ref, *scr):
        o_bufs, s_bufs = scr[0:4], scr[4:8]
        so, ro, ss, rs = scr[8:12], scr[12:16], scr[16:20], scr[20:24]

        my = lax.axis_index("i")
        left = (my - 1) % N_DEV
        right = (my + 1) % N_DEV

        barrier = pltpu.get_barrier_semaphore()
        for nbr in (left, right):
            pl.semaphore_signal(barrier, inc=1, device_id=(nbr,),
                                device_id_type=pl.DeviceIdType.MESH)
        pl.semaphore_wait(barrier, 2)

        rds = []
        for b in range(N_BLK):
            dst = right if b % 2 == 0 else left
            hops = []
            for h in range(N_DEV - 1):
                hops.append((
                    pltpu.make_async_remote_copy(
                        src_ref=o_bufs[b].at[h], dst_ref=o_bufs[b].at[h + 1],
                        send_sem=so[b].at[h], recv_sem=ro[b].at[h + 1],
                        device_id=(dst,), device_id_type=pl.DeviceIdType.MESH),
                    pltpu.make_async_remote_copy(
                        src_ref=s_bufs[b].at[h], dst_ref=s_bufs[b].at[h + 1],
                        send_sem=ss[b].at[h], recv_sem=rs[b].at[h + 1],
                        device_id=(dst,), device_id_type=pl.DeviceIdType.MESH),
                ))
            rds.append(hops)

        def start(b, h):
            rds[b][h][0].start()
            rds[b][h][1].start()

        def serve(b, h):
            rds[b][h][0].wait_recv()
            rds[b][h][1].wait_recv()
            if h + 1 < N_DEV - 1:
                start(b, h + 1)

        xb = x_ref[0].astype(bf16)
        wqb = wq_ref[...].astype(bf16)
        wob = wo_ref[...].astype(bf16)
        kb = [k_ref[0, :, g, :].astype(bf16) for g in range(HKV)]
        vb = [v_ref[0, :, g, :].astype(bf16) for g in range(HKV)]

        def compute_block(b):
            r0 = ROWS[b]
            q = jnp.dot(xb[r0:r0 + sqb], wqb,
                        preferred_element_type=f32)
            q = (q.reshape(sqb, HQ, DH) * SCALE).astype(bf16)
            o_h, m_h, l_h = [], [], []
            for h in range(HQ):
                s = lax.dot_general(q[:, h, :], kb[h // GQA],
                                    (((1,), (1,)), ((), ())),
                                    preferred_element_type=f32)
                mh = jnp.max(s, axis=1)
                p = jnp.exp(s - mh[:, None])
                lh = jnp.sum(p, axis=1)
                oh = jnp.dot(p.astype(bf16), vb[h // GQA],
                             preferred_element_type=f32)
                o_h.append(oh)
                m_h.append(mh)
                l_h.append(lh)
            o_bufs[b][0] = jnp.stack(o_h, axis=1).astype(bf16)
            s_bufs[b][0] = jnp.stack(
                [jnp.stack(m_h, axis=0), jnp.stack(l_h, axis=0)], axis=0
            )

        def merge_block(b):
            r0 = ROWS[b]
            cs = s_bufs[b][...]
            ms, ls = cs[:, 0], cs[:, 1]
            m_new = jnp.max(ms, axis=0)
            w = jnp.exp(ms - m_new[None])
            l_m = jnp.sum(ls * w, axis=0)
            co = o_bufs[b][...].astype(f32)
            w_t = jnp.transpose(w, (0, 2, 1))
            o_m = jnp.sum(co * w_t[..., None], axis=0)
            attn = (o_m / jnp.transpose(l_m)[..., None]).reshape(sqb, HQ * DH)
            out_ref[0, r0:r0 + sqb, :] = jnp.dot(
                attn.astype(bf16), wob, preferred_element_type=f32)

        compute_block(0); start(0, 0)
        compute_block(1); start(1, 0)
        compute_block(2); start(2, 0); serve(0, 0)
        compute_block(3); start(3, 0); serve(1, 0)
        serve(0, 1)
        serve(2, 0)
        serve(1, 1)
        serve(3, 0)
        serve(0, 2); merge_block(0)
        serve(2, 1)
        serve(1, 2); merge_block(1)
        serve(3, 1)
        serve(2, 2); merge_block(2)
        serve(3, 2); merge_block(3)

        for b in range(N_BLK):
            for h in range(N_DEV - 1):
                rds[b][h][0].wait_send()
                rds[b][h][1].wait_send()

    return pl.pallas_call(
        body,
        out_shape=jax.ShapeDtypeStruct((1, sq, d), jnp.float32),
        in_specs=[pl.BlockSpec(memory_space=pltpu.VMEM)] * 5,
        out_specs=pl.BlockSpec(memory_space=pltpu.VMEM),
        scratch_shapes=(
            [pltpu.VMEM((N_DEV, sqb, HQ, DH), jnp.bfloat16)] * N_BLK
            + [pltpu.VMEM((N_DEV, 2, HQ, sqb), jnp.float32)] * N_BLK
            + [pltpu.SemaphoreType.DMA((N_DEV,))] * (4 * N_BLK)
        ),
        compiler_params=pltpu.CompilerParams(
            collective_id=0, vmem_limit_bytes=100 * 1024 * 1024
        ),
    )(x, Wq, Wo, K_ext, V_ext)


# device time: 61847 ns/iter; 1.0396x vs baseline; 1.0396x over previous
import jax
import jax.numpy as jnp
from jax import lax
from jax.experimental import pallas as pl
from jax.experimental.pallas import tpu as pltpu

N_DEV = 4
N_GRP = 4
SCALE = 0.08838834764831843
HQ, HKV, DH = 8, 2, 128
HG = HQ // N_GRP
GQA = HQ // HKV


def kernel(x, Wq, Wo, K_ext, V_ext):
    _, sq, d = x.shape
    f32, bf16 = jnp.float32, jnp.bfloat16

    def body(x_ref, wq_ref, wo_ref, k_ref, v_ref, out_ref, *scr):
        o_bufs, s_bufs = scr[0:4], scr[4:8]
        so, ro, ss, rs = scr[8:12], scr[12:16], scr[16:20], scr[20:24]

        my = lax.axis_index("i")
        left = (my - 1) % N_DEV
        right = (my + 1) % N_DEV

        barrier = pltpu.get_barrier_semaphore()
        for nbr in (left, right):
            pl.semaphore_signal(barrier, inc=1, device_id=(nbr,),
                                device_id_type=pl.DeviceIdType.MESH)
        pl.semaphore_wait(barrier, 2)

        rds = []
        for g in range(N_GRP):
            dst = right if g % 2 == 0 else left
            hops = []
            for h in range(N_DEV - 1):
                hops.append((
                    pltpu.make_async_remote_copy(
                        src_ref=o_bufs[g].at[h], dst_ref=o_bufs[g].at[h + 1],
                        send_sem=so[g].at[h], recv_sem=ro[g].at[h + 1],
                        device_id=(dst,), device_id_type=pl.DeviceIdType.MESH),
                    pltpu.make_async_remote_copy(
                        src_ref=s_bufs[g].at[h], dst_ref=s_bufs[g].at[h + 1],
                        send_sem=ss[g].at[h], recv_sem=rs[g].at[h + 1],
                        device_id=(dst,), device_id_type=pl.DeviceIdType.MESH),
                ))
            rds.append(hops)

        def start(g, h):
            rds[g][h][0].start()
            rds[g][h][1].start()

        def serve(g, h):
            rds[g][h][0].wait_recv()
            rds[g][h][1].wait_recv()
            if h + 1 < N_DEV - 1:
                start(g, h + 1)

        xb = x_ref[0].astype(bf16)
        wqb = wq_ref[...].astype(bf16)
        kb = [k_ref[0, :, g, :].astype(bf16) for g in range(HKV)]
        vb = [v_ref[0, :, g, :].astype(bf16) for g in range(HKV)]

        q = jnp.dot(xb, wqb, preferred_element_type=f32)
        q = (q.reshape(sq, HQ, DH) * SCALE).astype(bf16)

        def compute_group(g):
            o_h, m_h, l_h = [], [], []
            for h in range(g * HG, (g + 1) * HG):
                s = lax.dot_general(q[:, h, :], kb[h // GQA],
                                    (((1,), (1,)), ((), ())),
                                    preferred_element_type=f32)
                mh = jnp.max(s, axis=1)
                p = jnp.exp(s - mh[:, None])
                lh = jnp.sum(p, axis=1)
                oh = jnp.dot(p.astype(bf16), vb[h // GQA],
                             preferred_element_type=f32)
                o_h.append(oh)
                m_h.append(mh)
                l_h.append(lh)
            o_bufs[g][0] = jnp.stack(o_h, axis=1).astype(bf16)
            s_bufs[g][0] = jnp.stack(
                [jnp.stack(m_h, axis=0), jnp.stack(l_h, axis=0)], axis=0
            )

        def merge_group(g, acc):
            cs = s_bufs[g][...]
            ms, ls = cs[:, 0], cs[:, 1]
            m_new = jnp.max(ms, axis=0)
            w = jnp.exp(ms - m_new[None])
            l_m = jnp.sum(ls * w, axis=0)
            co = o_bufs[g][...].astype(f32)
            w_t = jnp.transpose(w, (0, 2, 1))
            o_m = jnp.sum(co * w_t[..., None], axis=0)
            attn = (o_m / jnp.transpose(l_m)[..., None]).reshape(sq, HG * DH)
            wo_g = wo_ref[g * HG * DH:(g + 1) * HG * DH, :].astype(bf16)
            return acc + jnp.dot(attn.astype(bf16), wo_g,
                                 preferred_element_type=f32)

        acc = jnp.zeros((sq, d), dtype=f32)
        compute_group(0); start(0, 0)
        compute_group(1); start(1, 0)
        compute_group(2); start(2, 0); serve(0, 0)
        compute_group(3); start(3, 0); serve(1, 0)
        serve(0, 1)
        serve(2, 0)
        serve(1, 1)
        serve(3, 0)
        serve(0, 2); acc = merge_group(0, acc)
        serve(2, 1)
        serve(1, 2); acc = merge_group(1, acc)
        serve(3, 1)
        serve(2, 2); acc = merge_group(2, acc)
        serve(3, 2); acc = merge_group(3, acc)
        out_ref[0] = acc

        for g in range(N_GRP):
            for h in range(N_DEV - 1):
                rds[g][h][0].wait_send()
                rds[g][h][1].wait_send()

    return pl.pallas_call(
        body,
        out_shape=jax.ShapeDtypeStruct((1, sq, d), jnp.float32),
        in_specs=[pl.BlockSpec(memory_space=pltpu.VMEM)] * 5,
        out_specs=pl.BlockSpec(memory_space=pltpu.VMEM),
        scratch_shapes=(
            [pltpu.VMEM((N_DEV, sq, HG, DH), jnp.bfloat16)] * N_GRP
            + [pltpu.VMEM((N_DEV, 2, HG, sq), jnp.float32)] * N_GRP
            + [pltpu.SemaphoreType.DMA((N_DEV,))] * (4 * N_GRP)
        ),
        compiler_params=pltpu.CompilerParams(
            collective_id=0, vmem_limit_bytes=100 * 1024 * 1024
        ),
    )(x, Wq, Wo, K_ext, V_ext)
